# Initial kernel scaffold; baseline (speedup 1.0000x reference)
import jax
import jax.numpy as jnp
from jax.experimental import pallas as pl


def kernel(input, emb_table, align_w, pos_table):
    raise NotImplementedError("write your pallas kernel here")


# speedup vs baseline: 1.7292x; 1.7292x over previous
import functools

import jax
import jax.numpy as jnp
from jax import lax
from jax.experimental import pallas as pl
from jax.experimental.pallas import tpu as pltpu
from jax.experimental.pallas import tpu_sc as plsc


_STREAM = 128
_K = 8


@functools.lru_cache(maxsize=None)
def _sc_gather_fn(rows, d, nc, ns):
    nw = nc * ns
    rpw = rows // nw
    ch = _STREAM * _K
    n_outer = rpw // ch
    groups_per_worker = rpw // _STREAM

    mesh = plsc.VectorSubcoreMesh(core_axis_name="c", subcore_axis_name="s")

    @functools.partial(
        pl.kernel,
        mesh=mesh,
        out_type=jax.ShapeDtypeStruct((rows, d), jnp.float32),
        scratch_types=[
            pltpu.VMEM((_K, _STREAM), jnp.int32),
            pltpu.VMEM((ch, d), jnp.float32),
            pltpu.SemaphoreType.DMA,
        ],
        compiler_params=pltpu.CompilerParams(use_tc_tiling_on_sc=False),
    )
    def gather_kernel(table_hbm, idx_hbm, out_hbm, idx_v, rows_v, sem):
        wid = lax.axis_index("s") * nc + lax.axis_index("c")
        gbase = wid * groups_per_worker

        def body(i, carry):
            g0 = gbase + i * _K
            pltpu.sync_copy(idx_hbm.at[pl.ds(g0, _K)], idx_v)
            copies = [
                pltpu.async_copy(
                    table_hbm.at[idx_v.at[j]],
                    rows_v.at[pl.ds(j * _STREAM, _STREAM)],
                    sem,
                )
                for j in range(_K)
            ]
            for cp in copies:
                cp.wait()
            pltpu.sync_copy(rows_v, out_hbm.at[pl.ds(g0 * _STREAM, ch)])
            return carry

        lax.fori_loop(0, n_outer, body, 0)

    return gather_kernel


def _tc_body(x_ref, m_ref, w_ref, p_ref, o_ref):
    x = x_ref[...]
    w = w_ref[...]
    aligned = lax.dot_general(
        x, w, (((1,), (1,)), ((), ())), preferred_element_type=jnp.float32
    )
    mask = m_ref[...] != 0
    o_ref[...] = aligned + jnp.where(mask, p_ref[...], 0.0)


def _tc_project(gathered, ids_col, align_w, pos_tiled, blk_rows):
    rows, d = gathered.shape
    return pl.pallas_call(
        _tc_body,
        grid=(rows // blk_rows,),
        in_specs=[
            pl.BlockSpec((blk_rows, d), lambda i: (i, 0)),
            pl.BlockSpec((blk_rows, 1), lambda i: (i, 0)),
            pl.BlockSpec((d, d), lambda i: (0, 0)),
            pl.BlockSpec((blk_rows, d), lambda i: (0, 0)),
        ],
        out_specs=pl.BlockSpec((blk_rows, d), lambda i: (i, 0)),
        out_shape=jax.ShapeDtypeStruct((rows, d), jnp.float32),
    )(gathered, ids_col, align_w, pos_tiled)


def kernel(input, emb_table, align_w, pos_table):
    b, s = input.shape
    v, d = emb_table.shape
    rows = b * s

    ids = input.reshape(rows).astype(jnp.int32)

    info = plsc.get_sparse_core_info()
    nc, ns = info.num_cores, info.num_subcores

    gathered = _sc_gather_fn(rows, d, nc, ns)(
        emb_table, ids.reshape(rows // _STREAM, _STREAM)
    )

    bb = 8
    blk_rows = bb * s
    pos_tiled = jnp.tile(pos_table[1 : s + 1], (bb, 1))
    out_flat = _tc_project(
        gathered, ids.reshape(rows, 1), align_w, pos_tiled, blk_rows
    )
    return out_flat.reshape(b, s, d)


# speedup vs baseline: 2.4995x; 1.4454x over previous
import functools

import jax
import jax.numpy as jnp
from jax import lax
from jax.experimental import pallas as pl
from jax.experimental.pallas import tpu as pltpu
from jax.experimental.pallas import tpu_sc as plsc


_STREAM = 128
_CHUNK = 1600
_BLK_ROWS = 2 * _CHUNK


@functools.lru_cache(maxsize=None)
def _sc_gather_fn(rows, d, nc, ns):
    nw = nc * ns
    rpw = rows // nw
    n_chunks = rpw // _CHUNK
    n_full = _CHUNK // _STREAM
    tail = _CHUNK - n_full * _STREAM

    mesh = plsc.VectorSubcoreMesh(core_axis_name="c", subcore_axis_name="s")

    @functools.partial(
        pl.kernel,
        mesh=mesh,
        out_type=jax.ShapeDtypeStruct((rows // 2, 2 * d), jnp.float32),
        scratch_types=[
            pltpu.VMEM((_CHUNK,), jnp.int32),
            pltpu.VMEM((_CHUNK, d), jnp.float32),
            pltpu.SemaphoreType.DMA,
        ],
        compiler_params=pltpu.CompilerParams(use_tc_tiling_on_sc=False),
    )
    def gather_kernel(table_hbm, idx_hbm, out_hbm, idx_v, rows_v, sem):
        wid = lax.axis_index("s") * nc + lax.axis_index("c")

        def body(t, carry):
            c = wid * n_chunks + t
            f0 = c * _CHUNK
            pltpu.sync_copy(idx_hbm.at[pl.ds(f0, _CHUNK)], idx_v)
            copies = [
                pltpu.async_copy(
                    table_hbm.at[idx_v.at[pl.ds(j * _STREAM, _STREAM)]],
                    rows_v.at[pl.ds(j * _STREAM, _STREAM)],
                    sem,
                )
                for j in range(n_full)
            ]
            if tail:
                copies.append(
                    pltpu.async_copy(
                        table_hbm.at[idx_v.at[pl.ds(n_full * _STREAM, tail)]],
                        rows_v.at[pl.ds(n_full * _STREAM, tail)],
                        sem,
                    )
                )
            for cp in copies:
                cp.wait()
            half = c % 2
            p0 = (c // 2) * _CHUNK
            pltpu.sync_copy(
                rows_v, out_hbm.at[pl.ds(p0, _CHUNK), pl.ds(half * d, d)]
            )
            return carry

        lax.fori_loop(0, n_chunks, body, 0)

    return gather_kernel


def _tc_body(x_ref, ids_ref, w_ref, pos_ref, o_ref):
    x2 = x_ref[...]
    w = w_ref[...]
    ya = lax.dot_general(
        x2[:, :64], w, (((1,), (1,)), ((), ())),
        preferred_element_type=jnp.float32,
    )
    yb = lax.dot_general(
        x2[:, 64:], w, (((1,), (1,)), ((), ())),
        preferred_element_type=jnp.float32,
    )
    y = jnp.concatenate([ya, yb], axis=0)
    y3 = y.reshape(o_ref.shape)
    ids3 = lax.broadcast_in_dim(ids_ref[...], o_ref.shape, (0, 1))
    o_ref[...] = y3 + jnp.where(ids3 != 0, pos_ref[...], 0.0)


def _tc_project(gathered2, ids2d, align_w, pos3, b, s, d):
    rows = b * s
    nb = _BLK_ROWS // s
    return pl.pallas_call(
        _tc_body,
        grid=(rows // _BLK_ROWS,),
        in_specs=[
            pl.BlockSpec((_CHUNK, 2 * d), lambda i: (i, 0)),
            pl.BlockSpec((nb, s), lambda i: (i, 0)),
            pl.BlockSpec((d, d), lambda i: (0, 0)),
            pl.BlockSpec((1, s, d), lambda i: (0, 0, 0)),
        ],
        out_specs=pl.BlockSpec((nb, s, d), lambda i: (i, 0, 0)),
        out_shape=jax.ShapeDtypeStruct((b, s, d), jnp.float32),
    )(gathered2, ids2d, align_w, pos3)


def kernel(input, emb_table, align_w, pos_table):
    b, s = input.shape
    v, d = emb_table.shape
    rows = b * s

    ids = input.reshape(rows).astype(jnp.int32)

    info = plsc.get_sparse_core_info()
    nc, ns = info.num_cores, info.num_subcores

    gathered2 = _sc_gather_fn(rows, d, nc, ns)(emb_table, ids)

    pos3 = pos_table[1 : s + 1][None]
    return _tc_project(gathered2, input, align_w, pos3, b, s, d)
